# baseline (speedup 1.0000x reference)
import jax
import jax.numpy as jnp
from jax.experimental import pallas as pl


def _mm_block(a_ref, v_ref, o_ref):
    a = a_ref[...]
    o_ref[0] = jnp.dot(a, v_ref[0], preferred_element_type=jnp.float32)
    o_ref[1] = jnp.dot(a, v_ref[1], preferred_element_type=jnp.float32)


def kernel(matrix, matrix_batch):
    m, k = matrix.shape
    b, _, f = matrix_batch.shape

    bm = 256
    return pl.pallas_call(
        _mm_block,
        grid=(m // bm,),
        in_specs=[
            pl.BlockSpec((bm, k), lambda i: (i, 0)),
            pl.BlockSpec((b, k, f), lambda i: (0, 0, 0)),
        ],
        out_specs=pl.BlockSpec((b, bm, f), lambda i: (0, i, 0)),
        out_shape=jax.ShapeDtypeStruct((b, m, f), jnp.float32),
    )(matrix, matrix_batch)


# speedup vs baseline: 1.2507x; 1.2507x over previous
import jax
import jax.numpy as jnp
from jax.experimental import pallas as pl


def _mm_block(a_ref, v_ref, o_ref):
    o_ref[...] = jnp.dot(a_ref[...], v_ref[...],
                         preferred_element_type=jnp.float32)


def kernel(matrix, matrix_batch):
    m, k = matrix.shape
    b, _, f = matrix_batch.shape
    n = b * f
    vectors = jnp.swapaxes(matrix_batch, 0, 1).reshape(k, n)

    bm = 256
    out = pl.pallas_call(
        _mm_block,
        grid=(m // bm,),
        in_specs=[
            pl.BlockSpec((bm, k), lambda i: (i, 0)),
            pl.BlockSpec((k, n), lambda i: (0, 0)),
        ],
        out_specs=pl.BlockSpec((bm, n), lambda i: (i, 0)),
        out_shape=jax.ShapeDtypeStruct((m, n), jnp.float32),
    )(matrix, vectors)

    return jnp.swapaxes(out.reshape(m, b, f), 0, 1)
